# baseline (speedup 1.0000x reference)
import jax
import jax.numpy as jnp
from jax import lax
from jax.experimental import pallas as pl

_PROMPT_LENGTH = 20
_EMBED_DIM = 64
_SEQ_LEN = 220
_BATCH_GROUP = 64


def _body(idx_ref, tbl_ref, out_ref):
    g = idx_ref.shape[-1]
    idx = idx_ref[0]
    s = lax.broadcasted_iota(jnp.int32, (1, g), 1) % _SEQ_LEN
    idx = idx + jnp.where(s >= _PROMPT_LENGTH, _PROMPT_LENGTH, 0)
    onehot_t = (idx == lax.broadcasted_iota(
        jnp.int32, (2 * _PROMPT_LENGTH, g), 0)).astype(jnp.float32)
    rows = lax.dot_general(
        onehot_t, tbl_ref[...], (((0,), (0,)), ((), ())),
        preferred_element_type=jnp.float32)
    out_ref[...] = rows.reshape(_BATCH_GROUP, _SEQ_LEN, _EMBED_DIM)


def kernel(input, shared_weight, prompt_weight):
    batch, seq_len = input.shape
    g = _BATCH_GROUP * seq_len
    n_groups = batch // _BATCH_GROUP
    idx = input.astype(jnp.int32).reshape(n_groups, 1, g)
    tbl = jnp.concatenate(
        [prompt_weight, shared_weight[:_PROMPT_LENGTH]], axis=0)

    return pl.pallas_call(
        _body,
        grid=(n_groups,),
        in_specs=[
            pl.BlockSpec((1, 1, g), lambda i: (i, 0, 0)),
            pl.BlockSpec((2 * _PROMPT_LENGTH, _EMBED_DIM), lambda i: (0, 0)),
        ],
        out_specs=pl.BlockSpec(
            (_BATCH_GROUP, seq_len, _EMBED_DIM), lambda i: (i, 0, 0)),
        out_shape=jax.ShapeDtypeStruct(
            (batch, seq_len, _EMBED_DIM), jnp.float32),
    )(idx, tbl)


# speedup vs baseline: 1.0381x; 1.0381x over previous
import jax
import jax.numpy as jnp
from jax import lax
from jax.experimental import pallas as pl

_PROMPT_LENGTH = 20
_EMBED_DIM = 64
_SEQ_LEN = 220
_BATCH_GROUP = 128


def _body(idx_ref, tbl_ref, out_ref):
    g = idx_ref.shape[-1]
    idx = idx_ref[0]
    s = lax.broadcasted_iota(jnp.int32, (1, g), 1) % _SEQ_LEN
    idx = idx + jnp.where(s >= _PROMPT_LENGTH, _PROMPT_LENGTH, 0)
    onehot_t = (idx == lax.broadcasted_iota(
        jnp.int32, (2 * _PROMPT_LENGTH, g), 0)).astype(jnp.float32)
    rows = lax.dot_general(
        onehot_t, tbl_ref[...], (((0,), (0,)), ((), ())),
        preferred_element_type=jnp.float32)
    out_ref[...] = rows.reshape(_BATCH_GROUP, _SEQ_LEN, _EMBED_DIM)


def kernel(input, shared_weight, prompt_weight):
    batch, seq_len = input.shape
    g = _BATCH_GROUP * seq_len
    n_groups = batch // _BATCH_GROUP
    idx = input.astype(jnp.int32).reshape(n_groups, 1, g)
    tbl = jnp.concatenate(
        [prompt_weight, shared_weight[:_PROMPT_LENGTH]], axis=0)

    return pl.pallas_call(
        _body,
        grid=(n_groups,),
        in_specs=[
            pl.BlockSpec((1, 1, g), lambda i: (i, 0, 0)),
            pl.BlockSpec((2 * _PROMPT_LENGTH, _EMBED_DIM), lambda i: (0, 0)),
        ],
        out_specs=pl.BlockSpec(
            (_BATCH_GROUP, seq_len, _EMBED_DIM), lambda i: (i, 0, 0)),
        out_shape=jax.ShapeDtypeStruct(
            (batch, seq_len, _EMBED_DIM), jnp.float32),
    )(idx, tbl)
